# baseline (speedup 1.0000x reference)
import functools

import jax
import jax.numpy as jnp
from jax import lax
from jax.experimental import pallas as pl
from jax.experimental.pallas import tpu as pltpu
from jax.experimental.pallas import tpu_sc as plsc

B, N, D, H, C = 8, 4096, 1024, 512, 2
K = 2867
KPAD = 2896
NC, NS = 2, 16
NT = 512

GQ = 720
GREM = K - 3 * GQ
CH = 48
NFULL_Q = GQ // CH
NFULL_R = GREM // CH
REM = GREM - NFULL_R * CH


def _mlp_body(x_ref, w1_ref, b1_ref, wa_ref, ba_ref, out_ref):
    x = x_ref[0]
    h = lax.dot_general(x, w1_ref[...], (((1,), (1,)), ((), ())),
                        preferred_element_type=jnp.float32,
                        precision=lax.Precision.DEFAULT)
    h = jnp.maximum(h + b1_ref[...], 0.0)
    a = lax.dot_general(wa_ref[...], h, (((1,), (1,)), ((), ())),
                        preferred_element_type=jnp.float32,
                        precision=lax.Precision.DEFAULT)
    out_ref[0] = a + ba_ref[...]


def _mlp(features, W1, b1, Wa, ba):
    return pl.pallas_call(
        _mlp_body,
        grid=(B, N // NT),
        in_specs=[
            pl.BlockSpec((1, NT, D), lambda b, n: (b, n, 0)),
            pl.BlockSpec((H, D), lambda b, n: (0, 0)),
            pl.BlockSpec((1, H), lambda b, n: (0, 0)),
            pl.BlockSpec((C, H), lambda b, n: (0, 0)),
            pl.BlockSpec((C, 1), lambda b, n: (0, 0)),
        ],
        out_specs=pl.BlockSpec((1, C, NT), lambda b, n: (b, 0, n)),
        out_shape=jax.ShapeDtypeStruct((B, C, N), jnp.float32),
    )(features, W1, b1.reshape(1, H), Wa, ba.reshape(C, 1))


def _softmax_select_body(a_ref, comb_ref, thr_ref, need_ref):
    a = a_ref[...]
    m = jnp.max(a, axis=2, keepdims=True)
    e = jnp.exp(a - m)
    s = jnp.sum(e, axis=2, keepdims=True)
    comb = jnp.mean(e / s, axis=1)
    comb_ref[...] = comb

    lo = jnp.zeros((B, 1), jnp.int32)
    hi = jnp.full((B, 1), 0x7F000000, jnp.int32)

    def it(_, lh):
        lo, hi = lh
        mid = lo + (hi - lo) // 2
        midf = lax.bitcast_convert_type(mid, jnp.float32)
        cnt = jnp.sum((comb > midf).astype(jnp.int32), axis=1,
                      keepdims=True)
        ge = cnt >= K
        return jnp.where(ge, mid, lo), jnp.where(ge, hi, mid)

    _, hi = lax.fori_loop(0, 31, it, (lo, hi))
    thr = lax.bitcast_convert_type(hi, jnp.float32)
    cgt = jnp.sum((comb > thr).astype(jnp.int32), axis=1, keepdims=True)
    thr_ref[...] = jnp.broadcast_to(thr, (B, 16))
    need_ref[...] = jnp.broadcast_to(K - cgt, (B, 16))


def _softmax_select(a_t):
    return pl.pallas_call(
        _softmax_select_body,
        out_shape=(
            jax.ShapeDtypeStruct((B, N), jnp.float32),
            jax.ShapeDtypeStruct((B, 16), jnp.float32),
            jax.ShapeDtypeStruct((B, 16), jnp.int32),
        ),
    )(a_t)


def _topk_idx_body(comb_hbm, thr_hbm, need_hbm, idx_hbm, idxg_hbm,
                   comb_v, thr_v, need_v, idx_v, idxg_v):
    wid = lax.axis_index("s") * NC + lax.axis_index("c")

    @pl.when(wid < B)
    def _():
        b = wid
        pltpu.sync_copy(comb_hbm.at[b], comb_v)
        pltpu.sync_copy(thr_hbm.at[b], thr_v)
        pltpu.sync_copy(need_hbm.at[b], need_v)
        thr = thr_v[...]
        need = need_v[...]
        idx_v[pl.ds(KPAD - 16, 16)] = jnp.zeros((16,), jnp.int32)
        idx_v[pl.ds(KPAD - 32, 16)] = jnp.zeros((16,), jnp.int32)
        idxg_v[pl.ds(KPAD - 16, 16)] = jnp.zeros((16,), jnp.int32) + b * N
        idxg_v[pl.ds(KPAD - 32, 16)] = jnp.zeros((16,), jnp.int32) + b * N

        def chunk(i, carry):
            o, neq = carry
            v = comb_v[pl.ds(i * 16, 16)]
            gt = v > thr
            eq = v == thr
            eqc = jnp.where(eq, 1, 0)
            excl = plsc.cumsum(eqc) - eqc
            take = eq & ((neq + excl) < need)
            sel = gt | take
            ids = lax.iota(jnp.int32, 16) + i * 16
            plsc.store_compressed(idx_v.at[pl.ds(o, 16)], ids, mask=sel)
            plsc.store_compressed(idxg_v.at[pl.ds(o, 16)], ids + b * N,
                                  mask=sel)
            return (o + jnp.sum(jnp.where(sel, 1, 0)),
                    neq + jnp.sum(eqc))

        lax.fori_loop(0, N // 16, chunk, (0, 0))
        pltpu.sync_copy(idx_v, idx_hbm.at[b])
        pltpu.sync_copy(idxg_v, idxg_hbm.at[b])


def _topk_idx(combined, thrb, needb):
    mesh = plsc.VectorSubcoreMesh(core_axis_name="c", subcore_axis_name="s")
    call = pl.kernel(
        _topk_idx_body,
        out_type=(jax.ShapeDtypeStruct((B, KPAD), jnp.int32),
                  jax.ShapeDtypeStruct((B, KPAD), jnp.int32)),
        mesh=mesh,
        compiler_params=pltpu.CompilerParams(needs_layout_passes=False),
        scratch_types=[
            pltpu.VMEM((N,), jnp.float32),
            pltpu.VMEM((16,), jnp.float32),
            pltpu.VMEM((16,), jnp.int32),
            pltpu.VMEM((KPAD,), jnp.int32),
            pltpu.VMEM((KPAD,), jnp.int32),
        ],
    )
    return call(combined, thrb, needb)


NCHW = 15


def _gather_body(feat_hbm, idxgf_hbm, out_hbm, idxc_v, rows_v, sem, sem2):
    wid = lax.axis_index("s") * NC + lax.axis_index("c")
    b = wid // 4
    q = wid % 4
    wb = b * K + q * GQ

    def run_chunks(nch, tail):
        hs = []
        for i in range(nch + (1 if tail else 0)):
            if i >= 2:
                hs[i - 2].wait()
            buf = rows_v.at[pl.ds((i % 2) * CH, CH)]
            pltpu.sync_copy(
                idxgf_hbm.at[pl.ds(b * KPAD + q * GQ + i * CH, CH)],
                idxc_v)
            pltpu.async_copy(feat_hbm.at[idxc_v], buf, sem).wait()
            if tail and i == nch:
                hs.append(pltpu.async_copy(
                    buf.at[pl.ds(0, REM)],
                    out_hbm.at[pl.ds(wb + i * CH, REM)], sem2))
            else:
                hs.append(pltpu.async_copy(
                    buf, out_hbm.at[pl.ds(wb + i * CH, CH)], sem2))
        hs[-2].wait()
        hs[-1].wait()

    @pl.when(q < 3)
    def _():
        run_chunks(NCHW, False)

    @pl.when(q == 3)
    def _():
        run_chunks(NFULL_R, True)


def _gather(feat_flat, idxg):
    mesh = plsc.VectorSubcoreMesh(core_axis_name="c", subcore_axis_name="s")
    call = pl.kernel(
        _gather_body,
        out_type=jax.ShapeDtypeStruct((B * K, 1, D), jnp.float32),
        mesh=mesh,
        compiler_params=pltpu.CompilerParams(needs_layout_passes=False),
        scratch_types=[
            pltpu.VMEM((CH,), jnp.int32),
            pltpu.VMEM((2 * CH, 1, D), jnp.float32),
            pltpu.SemaphoreType.DMA,
            pltpu.SemaphoreType.DMA,
        ],
    )
    return call(feat_flat.reshape(B * N, 1, D), idxg.reshape(B * KPAD))


def kernel(features, W1, b1, Wa, ba):
    a_t = _mlp(features, W1, b1, Wa, ba)
    combined, thrb, needb = _softmax_select(a_t)
    idxp, idxg = _topk_idx(combined, thrb, needb)
    selected = _gather(features.reshape(B * N, D), idxg)
    return (selected.reshape(B, K, D), combined, idxp[:, :K])


# speedup vs baseline: 3.8893x; 3.8893x over previous
import functools

import jax
import jax.numpy as jnp
from jax import lax
from jax.experimental import pallas as pl
from jax.experimental.pallas import tpu as pltpu
from jax.experimental.pallas import tpu_sc as plsc

B, N, D, H, C = 8, 4096, 1024, 512, 2
K = 2867
KPAD = 2896
NC, NS = 2, 16
NT = 512

GQ = 720
GREM = K - 3 * GQ
CH = 48
NFULL_Q = GQ // CH
NFULL_R = GREM // CH
REM = GREM - NFULL_R * CH


def _mlp_body(x_ref, w1_ref, b1_ref, wa_ref, ba_ref, out_ref):
    x = x_ref[0]
    h = lax.dot_general(x, w1_ref[...], (((1,), (1,)), ((), ())),
                        preferred_element_type=jnp.float32,
                        precision=lax.Precision.DEFAULT)
    h = jnp.maximum(h + b1_ref[...], 0.0)
    a = lax.dot_general(wa_ref[...], h, (((1,), (1,)), ((), ())),
                        preferred_element_type=jnp.float32,
                        precision=lax.Precision.DEFAULT)
    out_ref[0] = a + ba_ref[...]


def _mlp(features, W1, b1, Wa, ba):
    return pl.pallas_call(
        _mlp_body,
        grid=(B, N // NT),
        in_specs=[
            pl.BlockSpec((1, NT, D), lambda b, n: (b, n, 0)),
            pl.BlockSpec((H, D), lambda b, n: (0, 0)),
            pl.BlockSpec((1, H), lambda b, n: (0, 0)),
            pl.BlockSpec((C, H), lambda b, n: (0, 0)),
            pl.BlockSpec((C, 1), lambda b, n: (0, 0)),
        ],
        out_specs=pl.BlockSpec((1, C, NT), lambda b, n: (b, 0, n)),
        out_shape=jax.ShapeDtypeStruct((B, C, N), jnp.float32),
    )(features, W1, b1.reshape(1, H), Wa, ba.reshape(C, 1))


def _softmax_select_body(a_ref, comb_ref, thr_ref, need_ref):
    a = a_ref[...]
    m = jnp.max(a, axis=2, keepdims=True)
    e = jnp.exp(a - m)
    s = jnp.sum(e, axis=2, keepdims=True)
    comb = jnp.mean(e / s, axis=1)
    comb_ref[...] = comb

    lo = jnp.zeros((B, 1), jnp.int32)
    hi = jnp.full((B, 1), 0x7F000000, jnp.int32)

    def it(_, lh):
        lo, hi = lh
        mid = lo + (hi - lo) // 2
        midf = lax.bitcast_convert_type(mid, jnp.float32)
        cnt = jnp.sum((comb > midf).astype(jnp.int32), axis=1,
                      keepdims=True)
        ge = cnt >= K
        return jnp.where(ge, mid, lo), jnp.where(ge, hi, mid)

    _, hi = lax.fori_loop(0, 31, it, (lo, hi))
    thr = lax.bitcast_convert_type(hi, jnp.float32)
    cgt = jnp.sum((comb > thr).astype(jnp.int32), axis=1, keepdims=True)
    thr_ref[...] = jnp.broadcast_to(thr, (B, 16))
    need_ref[...] = jnp.broadcast_to(K - cgt, (B, 16))


def _softmax_select(a_t):
    return pl.pallas_call(
        _softmax_select_body,
        out_shape=(
            jax.ShapeDtypeStruct((B, N), jnp.float32),
            jax.ShapeDtypeStruct((B, 16), jnp.float32),
            jax.ShapeDtypeStruct((B, 16), jnp.int32),
        ),
    )(a_t)


def _topk_idx_body(comb_hbm, thr_hbm, need_hbm, idx_hbm, idxg_hbm,
                   comb_v, thr_v, need_v, idx_v, idxg_v):
    wid = lax.axis_index("s") * NC + lax.axis_index("c")

    @pl.when(wid < B)
    def _():
        b = wid
        pltpu.sync_copy(comb_hbm.at[b], comb_v)
        pltpu.sync_copy(thr_hbm.at[b], thr_v)
        pltpu.sync_copy(need_hbm.at[b], need_v)
        thr = thr_v[...]
        need = need_v[...]
        idx_v[pl.ds(KPAD - 16, 16)] = jnp.zeros((16,), jnp.int32)
        idx_v[pl.ds(KPAD - 32, 16)] = jnp.zeros((16,), jnp.int32)
        idxg_v[pl.ds(KPAD - 16, 16)] = jnp.zeros((16,), jnp.int32) + b * N
        idxg_v[pl.ds(KPAD - 32, 16)] = jnp.zeros((16,), jnp.int32) + b * N

        def chunk(i, carry):
            o, neq = carry
            v = comb_v[pl.ds(i * 16, 16)]
            gt = v > thr
            eq = v == thr
            eqc = jnp.where(eq, 1, 0)
            excl = plsc.cumsum(eqc) - eqc
            take = eq & ((neq + excl) < need)
            sel = gt | take
            ids = lax.iota(jnp.int32, 16) + i * 16
            plsc.store_compressed(idx_v.at[pl.ds(o, 16)], ids, mask=sel)
            plsc.store_compressed(idxg_v.at[pl.ds(o, 16)], ids + b * N,
                                  mask=sel)
            return (o + jnp.sum(jnp.where(sel, 1, 0)),
                    neq + jnp.sum(eqc))

        lax.fori_loop(0, N // 16, chunk, (0, 0))
        pltpu.sync_copy(idx_v, idx_hbm.at[b])
        pltpu.sync_copy(idxg_v, idxg_hbm.at[b])


def _topk_idx(combined, thrb, needb):
    mesh = plsc.VectorSubcoreMesh(core_axis_name="c", subcore_axis_name="s")
    call = pl.kernel(
        _topk_idx_body,
        out_type=(jax.ShapeDtypeStruct((B, KPAD), jnp.int32),
                  jax.ShapeDtypeStruct((B, KPAD), jnp.int32)),
        mesh=mesh,
        compiler_params=pltpu.CompilerParams(needs_layout_passes=False),
        scratch_types=[
            pltpu.VMEM((N,), jnp.float32),
            pltpu.VMEM((16,), jnp.float32),
            pltpu.VMEM((16,), jnp.int32),
            pltpu.VMEM((KPAD,), jnp.int32),
            pltpu.VMEM((KPAD,), jnp.int32),
        ],
    )
    return call(combined, thrb, needb)


NCHW = 15
KOUT = 2880


def _gather_body(feat_hbm, idxgf_hbm, out_hbm, idxc_v, rows_v, sem, sem2):
    wid = lax.axis_index("s") * NC + lax.axis_index("c")
    b = wid // 4
    q = wid % 4
    base = q * (KOUT // 4)

    hs = []
    for i in range(NCHW):
        if i >= 2:
            hs[i - 2].wait()
        start = pl.multiple_of(base + i * CH, 8)
        buf = rows_v.at[pl.ds((i % 2) * CH, CH)]
        pltpu.sync_copy(idxgf_hbm.at[pl.ds(b * KPAD + start, CH)], idxc_v)
        pltpu.async_copy(feat_hbm.at[idxc_v], buf, sem).wait()
        hs.append(pltpu.async_copy(buf, out_hbm.at[b, pl.ds(start, CH)],
                                   sem2))
    hs[-2].wait()
    hs[-1].wait()


def _gather(feat_flat, idxg):
    mesh = plsc.VectorSubcoreMesh(core_axis_name="c", subcore_axis_name="s")
    call = pl.kernel(
        _gather_body,
        out_type=jax.ShapeDtypeStruct((B, KOUT, D), jnp.float32),
        mesh=mesh,
        compiler_params=pltpu.CompilerParams(needs_layout_passes=False),
        scratch_types=[
            pltpu.VMEM((CH,), jnp.int32),
            pltpu.VMEM((2 * CH, D), jnp.float32),
            pltpu.SemaphoreType.DMA,
            pltpu.SemaphoreType.DMA,
        ],
    )
    return call(feat_flat, idxg.reshape(B * KPAD))


def kernel(features, W1, b1, Wa, ba):
    a_t = _mlp(features, W1, b1, Wa, ba)
    combined, thrb, needb = _softmax_select(a_t)
    idxp, idxg = _topk_idx(combined, thrb, needb)
    selected = _gather(features.reshape(B * N, D), idxg)
    return (selected[:, :K], combined, idxp[:, :K])
